# baseline (speedup 1.0000x reference)
import functools

import jax
import jax.numpy as jnp
from jax import lax
from jax.experimental import pallas as pl
from jax.experimental.pallas import tpu as pltpu
from jax.experimental.pallas import tpu_sc as plsc

B = 4
S = 4096
H = 2048
NV = 1024
NJ = B * NV
NROWS = B * S
NC = 2
NS = 16
NW = NC * NS
RPW = NROWS // NW
MWORDS = RPW // 2 + 1
L0 = 272
LN = RPW + 16


def _body(text_hbm, vis_hbm, vidx_hbm, out_hbm,
          fidx_v, rbuf_v, sm, csem, gsem):
    wid = lax.axis_index("s") * NC + lax.axis_index("c")
    lo = wid * RPW

    cp = pltpu.async_copy(
        text_hbm.at[pl.ds(lo, RPW)], out_hbm.at[pl.ds(lo, RPW)], csem)

    pltpu.sync_copy(vidx_hbm, fidx_v)

    def init_step(i, carry):
        sm[i] = 0
        return carry
    lax.fori_loop(0, MWORDS, init_step, 0)

    def dedup_step(g, carry):
        v = fidx_v[pl.ds(g * 16, 16)]
        base = lax.shift_left(lax.shift_right_logical(g, 6), 12) - lo
        jbase = g * 16
        for l in range(16):
            idx = v[l]
            a = idx + base
            ok = jnp.logical_and(
                jnp.logical_and(a >= 0, a < RPW), idx != 0)
            a = lax.select(ok, a, RPW + 1)
            wi = lax.shift_right_logical(a, 1)
            half = lax.bitwise_and(a, 1)
            sh = lax.shift_left(half, 4)
            keep = lax.select(half == 1, jnp.int32(65535), jnp.int32(-65536))
            sm[wi] = lax.bitwise_or(
                lax.bitwise_and(sm[wi], keep),
                lax.shift_left(jbase + l + 1, sh))
        return carry
    lax.fori_loop(0, NJ // 16, dedup_step, 0)

    def compact_step(r, m):
        w = sm[lax.shift_right_logical(r, 1)]
        val = lax.bitwise_and(
            lax.shift_right_logical(
                w, lax.shift_left(lax.bitwise_and(r, 1), 4)),
            jnp.int32(65535))
        ok = val > 0
        q = lax.select(ok, m, RPW + lax.bitwise_and(r, 15))
        sm[L0 + q] = lax.bitwise_or(lax.shift_left(val - 1, 9), r)
        return m + lax.select(ok, 1, 0)
    m = lax.fori_loop(0, RPW, compact_step, 0)

    @pl.when(m > 0)
    def _pad():
        p0 = sm[L0]

        def pad_step(p, carry):
            sm[L0 + p] = p0
            return carry
        lax.fori_loop(m, lax.mul(lax.div(m + 15, 16), 16), pad_step, 0)

    cp.wait()

    nch = lax.div(m + 15, 16)

    iota = lax.iota(jnp.int32, 16)

    def chunk_step(c, carry):
        base = L0 + c * 16
        w = jnp.zeros((16,), jnp.int32)
        for l in range(16):
            w = jnp.where(iota == l, sm[base + l], w)
        svec = lax.shift_right_logical(w, 9)
        dvec = lax.bitwise_and(w, jnp.int32(RPW - 1)) + lo
        pltpu.async_copy(vis_hbm.at[svec], rbuf_v, gsem).wait()
        pltpu.sync_copy(rbuf_v, out_hbm.at[dvec])
        return carry
    lax.fori_loop(0, nch, chunk_step, 0)


@functools.partial(jax.jit, static_argnames=())
def kernel(vision_embeddings, text_embeddings, vision_indices):
    text_flat = jnp.reshape(text_embeddings, (NROWS, H))
    vidx_flat = jnp.reshape(vision_indices.astype(jnp.int32), (NJ,))

    out = pl.kernel(
        _body,
        out_type=jax.ShapeDtypeStruct((NROWS, H), jnp.float32),
        mesh=plsc.VectorSubcoreMesh(
            core_axis_name="c", subcore_axis_name="s",
            num_cores=NC, num_subcores=NS),
        scratch_types=[
            pltpu.VMEM((NJ,), jnp.int32),
            pltpu.VMEM((16, H), jnp.float32),
            pltpu.SMEM((1024,), jnp.int32),
            pltpu.SemaphoreType.DMA,
            pltpu.SemaphoreType.DMA,
        ],
    )(text_flat, vision_embeddings, vidx_flat)
    return jnp.reshape(out, (B, S, H))
